# baseline (speedup 1.0000x reference)
import functools

import jax
import jax.numpy as jnp
from jax import lax
from jax.experimental import pallas as pl
from jax.experimental.pallas import tpu as pltpu
from jax.experimental.pallas import tpu_sc as plsc

_N_EXC = 8000
_N = 10000
_NP = 10240
_E = 1_000_000
_T = 64
_B = 4
_DELAY = 15
_V_TH = 1.0
_DECAY = 0.9
_BETA = 10.0
_W_EXC = 0.1
_W_INH = -0.5

_COLS = 64
_CHUNK = 128
_NC, _NS = 2, 16
_NW = _NC * _NS
_NBUF = 4
_SEG = 64
_NSEG = 4
_CHUNKS_PER_W = _SEG * _NSEG
_E_PER_W = _CHUNKS_PER_W * _CHUNK
_EPAD = _E_PER_W * _NW
_DUMP_ROW = _N + 8

_ROWS_PER_S = _NP // _NS


def _spmv_body(src_hbm, dst_hbm, table_hbm, zero_hbm, out_hbm,
               src_v, dst_v, r0, r1, r2, r3, acc_sh,
               g0, g1, g2, g3, s0, s1, s2, s3):
    rows = [r0, r1, r2, r3]
    gsem = [g0, g1, g2, g3]
    ssem = [s0, s1, s2, s3]
    c = lax.axis_index("c")
    s = lax.axis_index("s")
    wid = c * _NS + s

    @pl.when(s == 0)
    def _():
        pltpu.sync_copy(zero_hbm, acc_sh)
    plsc.subcore_barrier()

    def seg_body(k, _):
        base = k * _SEG
        pltpu.sync_copy(src_hbm.at[wid].at[pl.ds(base, _SEG)], src_v)
        pltpu.sync_copy(dst_hbm.at[wid].at[pl.ds(base, _SEG)], dst_v)
        for b in range(_NBUF - 1):
            pltpu.async_copy(table_hbm.at[src_v.at[b]], rows[b], gsem[b])

        def body(i, _):
            for b in range(_NBUF):
                j = i * _NBUF + b
                bp = (b + _NBUF - 1) % _NBUF

                @pl.when(j > 0)
                def _(j=j, bp=bp):
                    pltpu.make_async_copy(
                        rows[bp], acc_sh.at[dst_v.at[j - 1]], ssem[bp]).wait()

                @pl.when(j + _NBUF - 1 < _SEG)
                def _(j=j, bp=bp):
                    pltpu.async_copy(
                        table_hbm.at[src_v.at[j + _NBUF - 1]], rows[bp],
                        gsem[bp])

                pltpu.make_async_copy(
                    table_hbm.at[src_v.at[j]], rows[b], gsem[b]).wait()
                pltpu.async_copy(rows[b], acc_sh.at[dst_v.at[j]], ssem[b],
                                 add=True)
            return ()

        lax.fori_loop(0, _SEG // _NBUF, body, ())
        pltpu.make_async_copy(
            rows[(_SEG - 1) % _NBUF], acc_sh.at[dst_v.at[_SEG - 1]],
            ssem[(_SEG - 1) % _NBUF]).wait()
        return ()

    lax.fori_loop(0, _NSEG, seg_body, ())
    plsc.subcore_barrier()

    r0 = s * _ROWS_PER_S
    pltpu.sync_copy(acc_sh.at[pl.ds(r0, _ROWS_PER_S)],
                    out_hbm.at[c].at[pl.ds(r0, _ROWS_PER_S)])


@functools.cache
def _get_spmv():
    return pl.kernel(
        _spmv_body,
        out_type=jax.ShapeDtypeStruct((_NC, _NP, _COLS), jnp.float32),
        mesh=plsc.VectorSubcoreMesh(core_axis_name="c", subcore_axis_name="s",
                                    num_cores=_NC, num_subcores=_NS),
        scratch_types=[
            pltpu.VMEM((_SEG, _CHUNK), jnp.int32),
            pltpu.VMEM((_SEG, _CHUNK), jnp.int32),
            pltpu.VMEM((_CHUNK, _COLS), jnp.float32),
            pltpu.VMEM((_CHUNK, _COLS), jnp.float32),
            pltpu.VMEM((_CHUNK, _COLS), jnp.float32),
            pltpu.VMEM((_CHUNK, _COLS), jnp.float32),
            pltpu.VMEM_SHARED((_NP, _COLS), jnp.float32),
        ] + [pltpu.SemaphoreType.DMA] * 8,
        compiler_params=pltpu.CompilerParams(use_tc_tiling_on_sc=False),
    )


def _lif_block_body(tb, irec_ref, ext_ref, vin_ref, spk_ref, vseq_ref, vout_ref):
    v = vin_ref[...]
    for t in range(tb):
        i_tot = irec_ref[t] + ext_ref[t]
        v = v * _DECAY + i_tot
        s = 1.0 / (1.0 + jnp.exp(-_BETA * (v - _V_TH)))
        spk_ref[t] = s
        v = v * (1.0 - s)
        vseq_ref[t] = v
    vout_ref[...] = v


_NT = 2048


def _lif_block(tb):
    grid = (_NP // _NT,)
    return pl.pallas_call(
        functools.partial(_lif_block_body, tb),
        grid=grid,
        in_specs=[
            pl.BlockSpec((tb, _B, _NT), lambda i: (0, 0, i)),
            pl.BlockSpec((tb, _B, _NT), lambda i: (0, 0, i)),
            pl.BlockSpec((_B, _NT), lambda i: (0, i)),
        ],
        out_specs=[
            pl.BlockSpec((tb, _B, _NT), lambda i: (0, 0, i)),
            pl.BlockSpec((tb, _B, _NT), lambda i: (0, 0, i)),
            pl.BlockSpec((_B, _NT), lambda i: (0, i)),
        ],
        out_shape=[
            jax.ShapeDtypeStruct((tb, _B, _NP), jnp.float32),
            jax.ShapeDtypeStruct((tb, _B, _NP), jnp.float32),
            jax.ShapeDtypeStruct((_B, _NP), jnp.float32),
        ],
    )


_lif15 = _lif_block(15)
_lif4 = _lif_block(4)


def _make_table(spk, wcol):
    tab = spk.transpose(2, 0, 1).reshape(_NP, 15 * _B) * wcol
    return jnp.pad(tab, ((0, 0), (0, _COLS - 15 * _B)))


def kernel(external_input, edge_index, edge_weight):
    del edge_weight
    ext = jnp.pad(external_input, ((0, 0), (0, 0), (0, _NP - _N)))
    pad_idx = jnp.full((_EPAD - _E,), _DUMP_ROW, jnp.int32)
    srcp = jnp.concatenate([edge_index[0], pad_idx])
    srcp = srcp.reshape(_NW, _CHUNKS_PER_W, _CHUNK)
    dstp = jnp.concatenate([edge_index[1], pad_idx])
    dstp = dstp.reshape(_NW, _CHUNKS_PER_W, _CHUNK)
    wcol = jnp.where(jnp.arange(_NP) < _N_EXC, _W_EXC, _W_INH)
    wcol = wcol.astype(jnp.float32)[:, None]
    zero_tab = jnp.zeros((_NP, _COLS), jnp.float32)

    v = jnp.zeros((_B, _NP), jnp.float32)
    spks, vs = [], []

    z_irec = jnp.zeros((15, _B, _NP), jnp.float32)
    s_blk, vseq, v = _lif15(z_irec, ext[0:15], v)
    spks.append(s_blk)
    vs.append(vseq)
    table = _make_table(s_blk, wcol)

    spmv = _get_spmv()
    for b in range(1, 4):
        parts = spmv(srcp, dstp, table, zero_tab)
        irec = (parts[0] + parts[1])[:, : 15 * _B]
        irec = irec.T.reshape(15, _B, _NP)
        s_blk, vseq, v = _lif15(irec, ext[15 * b : 15 * b + 15], v)
        spks.append(s_blk)
        vs.append(vseq)
        table = _make_table(s_blk, wcol)

    parts = spmv(srcp, dstp, table, zero_tab)
    irec4 = (parts[0] + parts[1])[:, : 4 * _B].T.reshape(4, _B, _NP)
    s_blk, vseq, v = _lif4(irec4, ext[60:64], v)
    spks.append(s_blk)
    vs.append(vseq)

    spikes = jnp.concatenate(spks)[:, :, :_N]
    vout = jnp.concatenate(vs)[:, :, :_N]
    return spikes, vout


# speedup vs baseline: 7.1596x; 7.1596x over previous
import functools

import jax
import jax.numpy as jnp
from jax import lax
from jax.experimental import pallas as pl
from jax.experimental.pallas import tpu as pltpu
from jax.experimental.pallas import tpu_sc as plsc

_N_EXC = 8000
_N = 10000
_NP = 10240
_E = 1_000_000
_T = 64
_B = 4
_DELAY = 15
_V_TH = 1.0
_DECAY = 0.9
_BETA = 10.0
_W_EXC = 0.1
_W_INH = -0.5

_COLS = 64
_CHUNK = 128
_NC, _NS = 2, 16
_NW = _NC * _NS
_NBUF = 4
_SEG = 64
_NSEG = 4
_CHUNKS_PER_W = _SEG * _NSEG
_E_PER_W = _CHUNKS_PER_W * _CHUNK
_EPAD = _E_PER_W * _NW
_DUMP_ROW = _N + 8

_ROWS_PER_S = _NP // _NS


def _spmv_body(src_hbm, dst_hbm, table_hbm, zero_hbm, out_hbm,
               src_v, dst_v, r0, r1, r2, r3, acc_sh,
               g0, g1, g2, g3, s0, s1, s2, s3):
    rows = [r0, r1, r2, r3]
    gsem = [g0, g1, g2, g3]
    ssem = [s0, s1, s2, s3]
    c = lax.axis_index("c")
    s = lax.axis_index("s")
    wid = c * _NS + s

    @pl.when(s == 0)
    def _():
        pltpu.sync_copy(zero_hbm, acc_sh)
    plsc.subcore_barrier()

    def seg_body(k, _):
        base = k * _SEG
        pltpu.sync_copy(src_hbm.at[wid].at[pl.ds(base, _SEG)], src_v)
        pltpu.sync_copy(dst_hbm.at[wid].at[pl.ds(base, _SEG)], dst_v)
        for b in range(_NBUF - 1):
            pltpu.async_copy(table_hbm.at[src_v.at[b]], rows[b], gsem[b])

        def body(i, _):
            for b in range(_NBUF):
                j = i * _NBUF + b
                bp = (b + _NBUF - 1) % _NBUF

                @pl.when(j > 0)
                def _(j=j, bp=bp):
                    pltpu.make_async_copy(
                        rows[bp], acc_sh.at[dst_v.at[j - 1]], ssem[bp]).wait()

                @pl.when(j + _NBUF - 1 < _SEG)
                def _(j=j, bp=bp):
                    pltpu.async_copy(
                        table_hbm.at[src_v.at[j + _NBUF - 1]], rows[bp],
                        gsem[bp])

                pltpu.make_async_copy(
                    table_hbm.at[src_v.at[j]], rows[b], gsem[b]).wait()
                pltpu.async_copy(rows[b], acc_sh.at[dst_v.at[j]], ssem[b],
                                 add=True)
            return ()

        lax.fori_loop(0, _SEG // _NBUF, body, ())
        pltpu.make_async_copy(
            rows[(_SEG - 1) % _NBUF], acc_sh.at[dst_v.at[_SEG - 1]],
            ssem[(_SEG - 1) % _NBUF]).wait()
        return ()

    lax.fori_loop(0, _NSEG, seg_body, ())
    plsc.subcore_barrier()

    r0 = s * _ROWS_PER_S
    pltpu.sync_copy(acc_sh.at[pl.ds(r0, _ROWS_PER_S)],
                    out_hbm.at[c].at[pl.ds(r0, _ROWS_PER_S)])


@functools.cache
def _get_spmv():
    return pl.kernel(
        _spmv_body,
        out_type=jax.ShapeDtypeStruct((_NC, _NP, _COLS), jnp.float32),
        mesh=plsc.VectorSubcoreMesh(core_axis_name="c", subcore_axis_name="s",
                                    num_cores=_NC, num_subcores=_NS),
        scratch_types=[
            pltpu.VMEM((_SEG, _CHUNK), jnp.int32),
            pltpu.VMEM((_SEG, _CHUNK), jnp.int32),
            pltpu.VMEM((_CHUNK, _COLS), jnp.float32),
            pltpu.VMEM((_CHUNK, _COLS), jnp.float32),
            pltpu.VMEM((_CHUNK, _COLS), jnp.float32),
            pltpu.VMEM((_CHUNK, _COLS), jnp.float32),
            pltpu.VMEM_SHARED((_NP, _COLS), jnp.float32),
        ] + [pltpu.SemaphoreType.DMA] * 8,
        compiler_params=pltpu.CompilerParams(use_tc_tiling_on_sc=False),
    )


def _lif_block_body(tb, irec_ref, ext_ref, vin_ref, spk_ref, vseq_ref, vout_ref):
    v = vin_ref[...]
    for t in range(tb):
        i_tot = irec_ref[t] + ext_ref[t]
        v = v * _DECAY + i_tot
        s = 1.0 / (1.0 + jnp.exp(-_BETA * (v - _V_TH)))
        spk_ref[t] = s
        v = v * (1.0 - s)
        vseq_ref[t] = v
    vout_ref[...] = v


_NT = 2048


def _lif_block(tb):
    grid = (_NP // _NT,)
    return pl.pallas_call(
        functools.partial(_lif_block_body, tb),
        grid=grid,
        in_specs=[
            pl.BlockSpec((tb, _B, _NT), lambda i: (0, 0, i)),
            pl.BlockSpec((tb, _B, _NT), lambda i: (0, 0, i)),
            pl.BlockSpec((_B, _NT), lambda i: (0, i)),
        ],
        out_specs=[
            pl.BlockSpec((tb, _B, _NT), lambda i: (0, 0, i)),
            pl.BlockSpec((tb, _B, _NT), lambda i: (0, 0, i)),
            pl.BlockSpec((_B, _NT), lambda i: (0, i)),
        ],
        out_shape=[
            jax.ShapeDtypeStruct((tb, _B, _NP), jnp.float32),
            jax.ShapeDtypeStruct((tb, _B, _NP), jnp.float32),
            jax.ShapeDtypeStruct((_B, _NP), jnp.float32),
        ],
    )


_lif15 = _lif_block(15)
_lif4 = _lif_block(4)


def _make_table(spk, wcol):
    tab = spk.transpose(2, 0, 1).reshape(_NP, 15 * _B) * wcol
    return jnp.pad(tab, ((0, 0), (0, _COLS - 15 * _B)))


def kernel(external_input, edge_index, edge_weight):
    del edge_weight
    ext = jnp.pad(external_input, ((0, 0), (0, 0), (0, _NP - _N)))
    pad_idx = (_N + jnp.arange(_EPAD - _E, dtype=jnp.int32) % (_NP - _N))
    srcp = jnp.concatenate([edge_index[0], pad_idx])
    srcp = srcp.reshape(_NW, _CHUNKS_PER_W, _CHUNK)
    dstp = jnp.concatenate([edge_index[1], pad_idx])
    dstp = dstp.reshape(_NW, _CHUNKS_PER_W, _CHUNK)
    wcol = jnp.where(jnp.arange(_NP) < _N_EXC, _W_EXC, _W_INH)
    wcol = wcol.astype(jnp.float32)[:, None]
    zero_tab = jnp.zeros((_NP, _COLS), jnp.float32)

    v = jnp.zeros((_B, _NP), jnp.float32)
    spks, vs = [], []

    z_irec = jnp.zeros((15, _B, _NP), jnp.float32)
    s_blk, vseq, v = _lif15(z_irec, ext[0:15], v)
    spks.append(s_blk)
    vs.append(vseq)
    table = _make_table(s_blk, wcol)

    spmv = _get_spmv()
    for b in range(1, 4):
        parts = spmv(srcp, dstp, table, zero_tab)
        irec = (parts[0] + parts[1])[:, : 15 * _B]
        irec = irec.T.reshape(15, _B, _NP)
        s_blk, vseq, v = _lif15(irec, ext[15 * b : 15 * b + 15], v)
        spks.append(s_blk)
        vs.append(vseq)
        table = _make_table(s_blk, wcol)

    parts = spmv(srcp, dstp, table, zero_tab)
    irec4 = (parts[0] + parts[1])[:, : 4 * _B].T.reshape(4, _B, _NP)
    s_blk, vseq, v = _lif4(irec4, ext[60:64], v)
    spks.append(s_blk)
    vs.append(vseq)

    spikes = jnp.concatenate(spks)[:, :, :_N]
    vout = jnp.concatenate(vs)[:, :, :_N]
    return spikes, vout


# speedup vs baseline: 7.6865x; 1.0736x over previous
import functools

import jax
import jax.numpy as jnp
from jax import lax
from jax.experimental import pallas as pl
from jax.experimental.pallas import tpu as pltpu
from jax.experimental.pallas import tpu_sc as plsc

_N_EXC = 8000
_N = 10000
_NP = 10240
_E = 1_000_000
_T = 64
_B = 4
_DELAY = 15
_V_TH = 1.0
_DECAY = 0.9
_BETA = 10.0
_W_EXC = 0.1
_W_INH = -0.5

_COLS = 64
_CHUNK = 128
_NC, _NS = 2, 16
_NW = _NC * _NS
_NBUF = 4
_SEG = 64
_NSEG = 4
_CHUNKS_PER_W = _SEG * _NSEG
_E_PER_W = _CHUNKS_PER_W * _CHUNK
_EPAD = _E_PER_W * _NW
_DUMP_ROW = _N + 8

_ROWS_PER_S = _NP // _NS


def _spmv_body(src_hbm, dst_hbm, table_hbm, zero_hbm, out_hbm,
               src_v, dst_v, r0, r1, r2, r3, acc_sh,
               g0, g1, g2, g3, s0, s1, s2, s3):
    rows = [r0, r1, r2, r3]
    gsem = [g0, g1, g2, g3]
    ssem = [s0, s1, s2, s3]
    c = lax.axis_index("c")
    s = lax.axis_index("s")
    wid = c * _NS + s

    @pl.when(s == 0)
    def _():
        pltpu.sync_copy(zero_hbm, acc_sh)
    plsc.subcore_barrier()

    def seg_body(k, _):
        base = k * _SEG
        pltpu.sync_copy(src_hbm.at[wid].at[pl.ds(base, _SEG)], src_v)
        pltpu.sync_copy(dst_hbm.at[wid].at[pl.ds(base, _SEG)], dst_v)
        for b in range(_NBUF - 1):
            pltpu.async_copy(table_hbm.at[src_v.at[b]], rows[b], gsem[b])

        def body(i, _):
            for b in range(_NBUF):
                j = i * _NBUF + b
                bp = (b + _NBUF - 1) % _NBUF

                @pl.when(j > 0)
                def _(j=j, bp=bp):
                    pltpu.make_async_copy(
                        rows[bp], acc_sh.at[dst_v.at[j - 1]], ssem[bp]).wait()

                @pl.when(j + _NBUF - 1 < _SEG)
                def _(j=j, bp=bp):
                    pltpu.async_copy(
                        table_hbm.at[src_v.at[j + _NBUF - 1]], rows[bp],
                        gsem[bp])

                pltpu.make_async_copy(
                    table_hbm.at[src_v.at[j]], rows[b], gsem[b]).wait()
                pltpu.async_copy(rows[b], acc_sh.at[dst_v.at[j]], ssem[b],
                                 add=True)
            return ()

        lax.fori_loop(0, _SEG // _NBUF, body, ())
        pltpu.make_async_copy(
            rows[(_SEG - 1) % _NBUF], acc_sh.at[dst_v.at[_SEG - 1]],
            ssem[(_SEG - 1) % _NBUF]).wait()
        return ()

    lax.fori_loop(0, _NSEG, seg_body, ())
    plsc.subcore_barrier()

    r0 = s * _ROWS_PER_S
    pltpu.sync_copy(acc_sh.at[pl.ds(r0, _ROWS_PER_S)],
                    out_hbm.at[c].at[pl.ds(r0, _ROWS_PER_S)])


@functools.cache
def _get_spmv():
    return pl.kernel(
        _spmv_body,
        out_type=jax.ShapeDtypeStruct((_NC, _NP, _COLS), jnp.float32),
        mesh=plsc.VectorSubcoreMesh(core_axis_name="c", subcore_axis_name="s",
                                    num_cores=_NC, num_subcores=_NS),
        scratch_types=[
            pltpu.VMEM((_SEG, _CHUNK), jnp.int32),
            pltpu.VMEM((_SEG, _CHUNK), jnp.int32),
            pltpu.VMEM((_CHUNK, _COLS), jnp.float32),
            pltpu.VMEM((_CHUNK, _COLS), jnp.float32),
            pltpu.VMEM((_CHUNK, _COLS), jnp.float32),
            pltpu.VMEM((_CHUNK, _COLS), jnp.float32),
            pltpu.VMEM_SHARED((_NP, _COLS), jnp.float32),
        ] + [pltpu.SemaphoreType.DMA] * 8,
        compiler_params=pltpu.CompilerParams(use_tc_tiling_on_sc=False),
    )


def _lif_block_body(tb, parts_ref, ext_ref, w_ref, vin_ref,
                    spk_ref, vseq_ref, vout_ref, tab_ref):
    irec = parts_ref[0] + parts_ref[1]
    irec_t = jnp.transpose(irec, (1, 0))
    w_row = w_ref[...]
    v = vin_ref[...]
    scaled = []
    for t in range(tb):
        i_tot = irec_t[4 * t : 4 * t + 4] + ext_ref[t]
        v = v * _DECAY + i_tot
        s = 1.0 / (1.0 + jnp.exp(-_BETA * (v - _V_TH)))
        spk_ref[t] = s
        v = v * (1.0 - s)
        vseq_ref[t] = v
        scaled.append(s * w_row)
    vout_ref[...] = v
    zpad = jnp.zeros((_COLS - 4 * tb, v.shape[-1]), jnp.float32)
    tab_ref[...] = jnp.transpose(
        jnp.concatenate(scaled + [zpad], axis=0), (1, 0))


_NT = 2048


def _lif_block(tb):
    grid = (_NP // _NT,)
    return pl.pallas_call(
        functools.partial(_lif_block_body, tb),
        grid=grid,
        in_specs=[
            pl.BlockSpec((_NC, _NT, _COLS), lambda i: (0, i, 0)),
            pl.BlockSpec((tb, _B, _NT), lambda i: (0, 0, i)),
            pl.BlockSpec((1, _NT), lambda i: (0, i)),
            pl.BlockSpec((_B, _NT), lambda i: (0, i)),
        ],
        out_specs=[
            pl.BlockSpec((tb, _B, _NT), lambda i: (0, 0, i)),
            pl.BlockSpec((tb, _B, _NT), lambda i: (0, 0, i)),
            pl.BlockSpec((_B, _NT), lambda i: (0, i)),
            pl.BlockSpec((_NT, _COLS), lambda i: (i, 0)),
        ],
        out_shape=[
            jax.ShapeDtypeStruct((tb, _B, _NP), jnp.float32),
            jax.ShapeDtypeStruct((tb, _B, _NP), jnp.float32),
            jax.ShapeDtypeStruct((_B, _NP), jnp.float32),
            jax.ShapeDtypeStruct((_NP, _COLS), jnp.float32),
        ],
    )


_lif15 = _lif_block(15)
_lif4 = _lif_block(4)


def kernel(external_input, edge_index, edge_weight):
    del edge_weight
    ext = jnp.pad(external_input, ((0, 0), (0, 0), (0, _NP - _N)))
    pad_idx = (_N + jnp.arange(_EPAD - _E, dtype=jnp.int32) % (_NP - _N))
    srcp = jnp.concatenate([edge_index[0], pad_idx])
    srcp = srcp.reshape(_NW, _CHUNKS_PER_W, _CHUNK)
    dstp = jnp.concatenate([edge_index[1], pad_idx])
    dstp = dstp.reshape(_NW, _CHUNKS_PER_W, _CHUNK)
    w_row = jnp.where(jnp.arange(_NP) < _N_EXC, _W_EXC, _W_INH)
    w_row = w_row.astype(jnp.float32)[None, :]
    zero_tab = jnp.zeros((_NP, _COLS), jnp.float32)
    zero_parts = jnp.zeros((_NC, _NP, _COLS), jnp.float32)

    v = jnp.zeros((_B, _NP), jnp.float32)
    spks, vs = [], []

    s_blk, vseq, v, table = _lif15(zero_parts, ext[0:15], w_row, v)
    spks.append(s_blk)
    vs.append(vseq)

    spmv = _get_spmv()
    for b in range(1, 4):
        parts = spmv(srcp, dstp, table, zero_tab)
        s_blk, vseq, v, table = _lif15(parts, ext[15 * b : 15 * b + 15],
                                       w_row, v)
        spks.append(s_blk)
        vs.append(vseq)

    parts = spmv(srcp, dstp, table, zero_tab)
    s_blk, vseq, v, _ = _lif4(parts, ext[60:64], w_row, v)
    spks.append(s_blk)
    vs.append(vseq)

    spikes = jnp.concatenate(spks)[:, :, :_N]
    vout = jnp.concatenate(vs)[:, :, :_N]
    return spikes, vout


# speedup vs baseline: 8.5331x; 1.1101x over previous
import functools

import jax
import jax.numpy as jnp
from jax import lax
from jax.experimental import pallas as pl
from jax.experimental.pallas import tpu as pltpu
from jax.experimental.pallas import tpu_sc as plsc

_N_EXC = 8000
_N = 10000
_NP = 10240
_E = 1_000_000
_T = 64
_B = 4
_DELAY = 15
_V_TH = 1.0
_DECAY = 0.9
_BETA = 10.0
_W_EXC = 0.1
_W_INH = -0.5

_COLS = 64
_CHUNK = 128
_NC, _NS = 2, 16
_NW = _NC * _NS
_NBUF = 4
_SEG = 64
_SEG_SIZES = (64, 64, 64, 56)
_CHUNKS_PER_W = sum(_SEG_SIZES)
_E_PER_W = _CHUNKS_PER_W * _CHUNK
_EPAD = _E_PER_W * _NW
_DUMP_ROW = _N + 8

_ROWS_PER_S = _NP // _NS


def _spmv_body(src_hbm, dst_hbm, table_hbm, zero_hbm, out_hbm,
               src_v, dst_v, r0, r1, r2, r3, acc_sh,
               g0, g1, g2, g3, s0, s1, s2, s3):
    rows = [r0, r1, r2, r3]
    gsem = [g0, g1, g2, g3]
    ssem = [s0, s1, s2, s3]
    c = lax.axis_index("c")
    s = lax.axis_index("s")
    wid = c * _NS + s

    @pl.when(s == 0)
    def _():
        pltpu.sync_copy(zero_hbm, acc_sh)
    plsc.subcore_barrier()

    base = 0
    for nch in _SEG_SIZES:
        pltpu.sync_copy(src_hbm.at[wid].at[pl.ds(base, nch)],
                        src_v.at[pl.ds(0, nch)])
        pltpu.sync_copy(dst_hbm.at[wid].at[pl.ds(base, nch)],
                        dst_v.at[pl.ds(0, nch)])
        for b in range(_NBUF - 1):
            pltpu.async_copy(table_hbm.at[src_v.at[b]], rows[b], gsem[b])

        def body(i, _, nch=nch):
            for b in range(_NBUF):
                j = i * _NBUF + b
                bp = (b + _NBUF - 1) % _NBUF

                @pl.when(j > 0)
                def _(j=j, bp=bp):
                    pltpu.make_async_copy(
                        rows[bp], acc_sh.at[dst_v.at[j - 1]], ssem[bp]).wait()

                @pl.when(j + _NBUF - 1 < nch)
                def _(j=j, bp=bp):
                    pltpu.async_copy(
                        table_hbm.at[src_v.at[j + _NBUF - 1]], rows[bp],
                        gsem[bp])

                pltpu.make_async_copy(
                    table_hbm.at[src_v.at[j]], rows[b], gsem[b]).wait()
                pltpu.async_copy(rows[b], acc_sh.at[dst_v.at[j]], ssem[b],
                                 add=True)
            return ()

        lax.fori_loop(0, nch // _NBUF, body, ())
        pltpu.make_async_copy(
            rows[(nch - 1) % _NBUF], acc_sh.at[dst_v.at[nch - 1]],
            ssem[(nch - 1) % _NBUF]).wait()
        base += nch

    plsc.subcore_barrier()

    r0 = s * _ROWS_PER_S
    pltpu.sync_copy(acc_sh.at[pl.ds(r0, _ROWS_PER_S)],
                    out_hbm.at[c].at[pl.ds(r0, _ROWS_PER_S)])


@functools.cache
def _get_spmv(cols):
    return pl.kernel(
        _spmv_body,
        out_type=jax.ShapeDtypeStruct((_NC, _NP, cols), jnp.float32),
        mesh=plsc.VectorSubcoreMesh(core_axis_name="c", subcore_axis_name="s",
                                    num_cores=_NC, num_subcores=_NS),
        scratch_types=[
            pltpu.VMEM((_SEG, _CHUNK), jnp.int32),
            pltpu.VMEM((_SEG, _CHUNK), jnp.int32),
            pltpu.VMEM((_CHUNK, cols), jnp.float32),
            pltpu.VMEM((_CHUNK, cols), jnp.float32),
            pltpu.VMEM((_CHUNK, cols), jnp.float32),
            pltpu.VMEM((_CHUNK, cols), jnp.float32),
            pltpu.VMEM_SHARED((_NP, cols), jnp.float32),
        ] + [pltpu.SemaphoreType.DMA] * 8,
        compiler_params=pltpu.CompilerParams(use_tc_tiling_on_sc=False),
    )


def _lif_block_body(tb, emit_tab, parts_ref, ext_ref, w_ref, vin_ref,
                    spk_ref, vseq_ref, vout_ref, tab_ref=None):
    irec = parts_ref[0] + parts_ref[1]
    irec_t = jnp.transpose(irec, (1, 0))
    w_row = w_ref[...]
    v = vin_ref[...]
    scaled = []
    for t in range(tb):
        i_tot = irec_t[4 * t : 4 * t + 4] + ext_ref[t]
        v = v * _DECAY + i_tot
        s = 1.0 / (1.0 + jnp.exp(-_BETA * (v - _V_TH)))
        spk_ref[t] = s
        v = v * (1.0 - s)
        vseq_ref[t] = v
        scaled.append(s * w_row)
    vout_ref[...] = v
    if emit_tab:
        zpad = jnp.zeros((_COLS - 4 * tb, v.shape[-1]), jnp.float32)
        tab_ref[...] = jnp.transpose(
            jnp.concatenate(scaled + [zpad], axis=0), (1, 0))


_NT = 2048


def _lif_block(tb, pcols, emit_tab):
    grid = (_NP // _NT,)
    out_specs = [
        pl.BlockSpec((tb, _B, _NT), lambda i: (0, 0, i)),
        pl.BlockSpec((tb, _B, _NT), lambda i: (0, 0, i)),
        pl.BlockSpec((_B, _NT), lambda i: (0, i)),
    ]
    out_shape = [
        jax.ShapeDtypeStruct((tb, _B, _NP), jnp.float32),
        jax.ShapeDtypeStruct((tb, _B, _NP), jnp.float32),
        jax.ShapeDtypeStruct((_B, _NP), jnp.float32),
    ]
    if emit_tab:
        out_specs.append(pl.BlockSpec((_NT, _COLS), lambda i: (i, 0)))
        out_shape.append(jax.ShapeDtypeStruct((_NP, _COLS), jnp.float32))
    return pl.pallas_call(
        functools.partial(_lif_block_body, tb, emit_tab),
        grid=grid,
        in_specs=[
            pl.BlockSpec((_NC, _NT, pcols), lambda i: (0, i, 0)),
            pl.BlockSpec((tb, _B, _NT), lambda i: (0, 0, i)),
            pl.BlockSpec((1, _NT), lambda i: (0, i)),
            pl.BlockSpec((_B, _NT), lambda i: (0, i)),
        ],
        out_specs=out_specs,
        out_shape=out_shape,
    )


_lif15 = _lif_block(15, _COLS, True)
_lif4 = _lif_block(4, 16, False)


def kernel(external_input, edge_index, edge_weight):
    del edge_weight
    ext = jnp.pad(external_input, ((0, 0), (0, 0), (0, _NP - _N)))
    pad_idx = (_N + jnp.arange(_EPAD - _E, dtype=jnp.int32) % (_NP - _N))
    srcp = jnp.concatenate([edge_index[0], pad_idx])
    srcp = srcp.reshape(_NW, _CHUNKS_PER_W, _CHUNK)
    dstp = jnp.concatenate([edge_index[1], pad_idx])
    dstp = dstp.reshape(_NW, _CHUNKS_PER_W, _CHUNK)
    w_row = jnp.where(jnp.arange(_NP) < _N_EXC, _W_EXC, _W_INH)
    w_row = w_row.astype(jnp.float32)[None, :]
    zero_tab = jnp.zeros((_NP, _COLS), jnp.float32)
    zero_tab16 = jnp.zeros((_NP, 16), jnp.float32)
    zero_parts = jnp.zeros((_NC, _NP, _COLS), jnp.float32)

    v = jnp.zeros((_B, _NP), jnp.float32)
    spks, vs = [], []

    s_blk, vseq, v, table = _lif15(zero_parts, ext[0:15], w_row, v)
    spks.append(s_blk)
    vs.append(vseq)

    spmv = _get_spmv(_COLS)
    for b in range(1, 4):
        parts = spmv(srcp, dstp, table, zero_tab)
        s_blk, vseq, v, table = _lif15(parts, ext[15 * b : 15 * b + 15],
                                       w_row, v)
        spks.append(s_blk)
        vs.append(vseq)

    parts = _get_spmv(16)(srcp, dstp, table[:, :16], zero_tab16)
    s_blk, vseq, v = _lif4(parts, ext[60:64], w_row, v)
    spks.append(s_blk)
    vs.append(vseq)

    spikes = jnp.concatenate(spks)[:, :, :_N]
    vout = jnp.concatenate(vs)[:, :, :_N]
    return spikes, vout


# speedup vs baseline: 8.6075x; 1.0087x over previous
import functools

import jax
import jax.numpy as jnp
from jax import lax
from jax.experimental import pallas as pl
from jax.experimental.pallas import tpu as pltpu
from jax.experimental.pallas import tpu_sc as plsc

_N_EXC = 8000
_N = 10000
_NP = 10240
_E = 1_000_000
_T = 64
_B = 4
_DELAY = 15
_V_TH = 1.0
_DECAY = 0.9
_BETA = 10.0
_W_EXC = 0.1
_W_INH = -0.5

_COLS = 64
_CHUNK = 128
_NC, _NS = 2, 16
_NW = _NC * _NS
_NBUF = 6
_PFD = 3
_SEG = 66
_SEG_SIZES = (60, 60, 60, 66)
_CHUNKS_PER_W = sum(_SEG_SIZES)
_E_PER_W = _CHUNKS_PER_W * _CHUNK
_EPAD = _E_PER_W * _NW
_DUMP_ROW = _N + 8

_ROWS_PER_S = _NP // _NS


def _spmv_body(src_hbm, dst_hbm, table_hbm, zero_hbm, out_hbm,
               src_v, dst_v, r0, r1, r2, r3, r4, r5, acc_sh,
               g0, g1, g2, g3, g4, g5, s0, s1, s2, s3, s4, s5):
    rows = [r0, r1, r2, r3, r4, r5]
    gsem = [g0, g1, g2, g3, g4, g5]
    ssem = [s0, s1, s2, s3, s4, s5]
    c = lax.axis_index("c")
    s = lax.axis_index("s")
    wid = c * _NS + s

    r0_ = s * _ROWS_PER_S
    pltpu.sync_copy(zero_hbm.at[pl.ds(r0_, _ROWS_PER_S)],
                    acc_sh.at[pl.ds(r0_, _ROWS_PER_S)])
    plsc.subcore_barrier()

    sdepth = _NBUF - _PFD
    base = 0
    for nch in _SEG_SIZES:
        pltpu.sync_copy(src_hbm.at[wid].at[pl.ds(base, nch)],
                        src_v.at[pl.ds(0, nch)])
        pltpu.sync_copy(dst_hbm.at[wid].at[pl.ds(base, nch)],
                        dst_v.at[pl.ds(0, nch)])
        for b in range(_PFD):
            pltpu.async_copy(table_hbm.at[src_v.at[b]], rows[b], gsem[b])

        def body(i, _, nch=nch):
            for b in range(_NBUF):
                j = i * _NBUF + b
                bf = (b + _PFD) % _NBUF

                @pl.when(j >= sdepth)
                def _(j=j, bf=bf):
                    pltpu.make_async_copy(
                        rows[bf], acc_sh.at[dst_v.at[j - sdepth]],
                        ssem[bf]).wait()

                @pl.when(j + _PFD < nch)
                def _(j=j, bf=bf):
                    pltpu.async_copy(
                        table_hbm.at[src_v.at[j + _PFD]], rows[bf], gsem[bf])

                pltpu.make_async_copy(
                    table_hbm.at[src_v.at[j]], rows[b], gsem[b]).wait()
                pltpu.async_copy(rows[b], acc_sh.at[dst_v.at[j]], ssem[b],
                                 add=True)
            return ()

        lax.fori_loop(0, nch // _NBUF, body, ())
        for k in range(sdepth):
            jj = nch - sdepth + k
            pltpu.make_async_copy(
                rows[jj % _NBUF], acc_sh.at[dst_v.at[jj]],
                ssem[jj % _NBUF]).wait()
        base += nch

    plsc.subcore_barrier()

    r0 = s * _ROWS_PER_S
    pltpu.sync_copy(acc_sh.at[pl.ds(r0, _ROWS_PER_S)],
                    out_hbm.at[c].at[pl.ds(r0, _ROWS_PER_S)])


@functools.cache
def _get_spmv(cols):
    return pl.kernel(
        _spmv_body,
        out_type=jax.ShapeDtypeStruct((_NC, _NP, cols), jnp.float32),
        mesh=plsc.VectorSubcoreMesh(core_axis_name="c", subcore_axis_name="s",
                                    num_cores=_NC, num_subcores=_NS),
        scratch_types=[
            pltpu.VMEM((_SEG, _CHUNK), jnp.int32),
            pltpu.VMEM((_SEG, _CHUNK), jnp.int32),
        ] + [pltpu.VMEM((_CHUNK, cols), jnp.float32)] * _NBUF + [
            pltpu.VMEM_SHARED((_NP, cols), jnp.float32),
        ] + [pltpu.SemaphoreType.DMA] * (2 * _NBUF),
        compiler_params=pltpu.CompilerParams(use_tc_tiling_on_sc=False),
    )


def _lif_block_body(tb, emit_tab, parts_ref, ext_ref, w_ref, vin_ref,
                    spk_ref, vseq_ref, vout_ref, tab_ref=None):
    irec = parts_ref[0] + parts_ref[1]
    irec_t = jnp.transpose(irec, (1, 0))
    w_row = w_ref[...]
    v = vin_ref[...]
    scaled = []
    for t in range(tb):
        i_tot = irec_t[4 * t : 4 * t + 4] + ext_ref[t]
        v = v * _DECAY + i_tot
        s = 1.0 / (1.0 + jnp.exp(-_BETA * (v - _V_TH)))
        spk_ref[t] = s
        v = v * (1.0 - s)
        vseq_ref[t] = v
        scaled.append(s * w_row)
    vout_ref[...] = v
    if emit_tab:
        zpad = jnp.zeros((_COLS - 4 * tb, v.shape[-1]), jnp.float32)
        tab_ref[...] = jnp.transpose(
            jnp.concatenate(scaled + [zpad], axis=0), (1, 0))


_NT = 2048


def _lif_block(tb, pcols, emit_tab):
    grid = (_NP // _NT,)
    out_specs = [
        pl.BlockSpec((tb, _B, _NT), lambda i: (0, 0, i)),
        pl.BlockSpec((tb, _B, _NT), lambda i: (0, 0, i)),
        pl.BlockSpec((_B, _NT), lambda i: (0, i)),
    ]
    out_shape = [
        jax.ShapeDtypeStruct((tb, _B, _NP), jnp.float32),
        jax.ShapeDtypeStruct((tb, _B, _NP), jnp.float32),
        jax.ShapeDtypeStruct((_B, _NP), jnp.float32),
    ]
    if emit_tab:
        out_specs.append(pl.BlockSpec((_NT, _COLS), lambda i: (i, 0)))
        out_shape.append(jax.ShapeDtypeStruct((_NP, _COLS), jnp.float32))
    return pl.pallas_call(
        functools.partial(_lif_block_body, tb, emit_tab),
        grid=grid,
        in_specs=[
            pl.BlockSpec((_NC, _NT, pcols), lambda i: (0, i, 0)),
            pl.BlockSpec((tb, _B, _NT), lambda i: (0, 0, i)),
            pl.BlockSpec((1, _NT), lambda i: (0, i)),
            pl.BlockSpec((_B, _NT), lambda i: (0, i)),
        ],
        out_specs=out_specs,
        out_shape=out_shape,
    )


_lif15 = _lif_block(15, _COLS, True)
_lif4 = _lif_block(4, 16, False)


def kernel(external_input, edge_index, edge_weight):
    del edge_weight
    ext = jnp.pad(external_input, ((0, 0), (0, 0), (0, _NP - _N)))
    pad_idx = (_N + jnp.arange(_EPAD - _E, dtype=jnp.int32) % (_NP - _N))
    srcp = jnp.concatenate([edge_index[0], pad_idx])
    srcp = srcp.reshape(_NW, _CHUNKS_PER_W, _CHUNK)
    dstp = jnp.concatenate([edge_index[1], pad_idx])
    dstp = dstp.reshape(_NW, _CHUNKS_PER_W, _CHUNK)
    w_row = jnp.where(jnp.arange(_NP) < _N_EXC, _W_EXC, _W_INH)
    w_row = w_row.astype(jnp.float32)[None, :]
    zero_tab = jnp.zeros((_NP, _COLS), jnp.float32)
    zero_tab16 = jnp.zeros((_NP, 16), jnp.float32)
    zero_parts = jnp.zeros((_NC, _NP, _COLS), jnp.float32)

    v = jnp.zeros((_B, _NP), jnp.float32)
    spks, vs = [], []

    s_blk, vseq, v, table = _lif15(zero_parts, ext[0:15], w_row, v)
    spks.append(s_blk)
    vs.append(vseq)

    spmv = _get_spmv(_COLS)
    for b in range(1, 4):
        parts = spmv(srcp, dstp, table, zero_tab)
        s_blk, vseq, v, table = _lif15(parts, ext[15 * b : 15 * b + 15],
                                       w_row, v)
        spks.append(s_blk)
        vs.append(vseq)

    parts = _get_spmv(16)(srcp, dstp, table[:, :16], zero_tab16)
    s_blk, vseq, v = _lif4(parts, ext[60:64], w_row, v)
    spks.append(s_blk)
    vs.append(vseq)

    spikes = jnp.concatenate(spks)[:, :, :_N]
    vout = jnp.concatenate(vs)[:, :, :_N]
    return spikes, vout


# speedup vs baseline: 8.7806x; 1.0201x over previous
import functools

import jax
import jax.numpy as jnp
from jax import lax
from jax.experimental import pallas as pl
from jax.experimental.pallas import tpu as pltpu
from jax.experimental.pallas import tpu_sc as plsc

_N_EXC = 8000
_N = 10000
_NP = 10240
_E = 1_000_000
_T = 64
_B = 4
_DELAY = 15
_V_TH = 1.0
_DECAY = 0.9
_BETA = 10.0
_W_EXC = 0.1
_W_INH = -0.5

_COLS = 64
_CHUNK = 128
_NC, _NS = 2, 16
_NW = _NC * _NS
_NBUF = 6
_PFD = 3
_SEG = 66
_SEG_SIZES = (60, 60, 60, 66)
_CHUNKS_PER_W = sum(_SEG_SIZES)
_E_PER_W = _CHUNKS_PER_W * _CHUNK
_EPAD = _E_PER_W * _NW
_DUMP_ROW = _N + 8

_ROWS_PER_S = _NP // _NS


def _spmv_body(src_hbm, dst_hbm, table_hbm, zero_hbm, out_hbm,
               src_v, dst_v, r0, r1, r2, r3, r4, r5, acc_sh,
               g0, g1, g2, g3, g4, g5, s0, s1, s2, s3, s4, s5):
    rows = [r0, r1, r2, r3, r4, r5]
    gsem = [g0, g1, g2, g3, g4, g5]
    ssem = [s0, s1, s2, s3, s4, s5]
    c = lax.axis_index("c")
    s = lax.axis_index("s")
    wid = c * _NS + s

    r0_ = s * _ROWS_PER_S
    pltpu.sync_copy(zero_hbm.at[pl.ds(r0_, _ROWS_PER_S)],
                    acc_sh.at[pl.ds(r0_, _ROWS_PER_S)])
    plsc.subcore_barrier()

    sdepth = _NBUF - _PFD
    base = 0
    for nch in _SEG_SIZES:
        pltpu.sync_copy(src_hbm.at[wid].at[pl.ds(base, nch)],
                        src_v.at[pl.ds(0, nch)])
        pltpu.sync_copy(dst_hbm.at[wid].at[pl.ds(base, nch)],
                        dst_v.at[pl.ds(0, nch)])
        for b in range(_PFD):
            pltpu.async_copy(table_hbm.at[src_v.at[b]], rows[b], gsem[b])

        def body(i, _, nch=nch):
            for b in range(_NBUF):
                j = i * _NBUF + b
                bf = (b + _PFD) % _NBUF

                @pl.when(j >= sdepth)
                def _(j=j, bf=bf):
                    pltpu.make_async_copy(
                        rows[bf], acc_sh.at[dst_v.at[j - sdepth]],
                        ssem[bf]).wait()

                @pl.when(j + _PFD < nch)
                def _(j=j, bf=bf):
                    pltpu.async_copy(
                        table_hbm.at[src_v.at[j + _PFD]], rows[bf], gsem[bf])

                pltpu.make_async_copy(
                    table_hbm.at[src_v.at[j]], rows[b], gsem[b]).wait()
                pltpu.async_copy(rows[b], acc_sh.at[dst_v.at[j]], ssem[b],
                                 add=True)
            return ()

        lax.fori_loop(0, nch // _NBUF, body, ())
        for k in range(sdepth):
            jj = nch - sdepth + k
            pltpu.make_async_copy(
                rows[jj % _NBUF], acc_sh.at[dst_v.at[jj]],
                ssem[jj % _NBUF]).wait()
        base += nch

    plsc.subcore_barrier()

    r0 = s * _ROWS_PER_S
    pltpu.sync_copy(acc_sh.at[pl.ds(r0, _ROWS_PER_S)],
                    out_hbm.at[c].at[pl.ds(r0, _ROWS_PER_S)])


@functools.cache
def _get_spmv(cols):
    return pl.kernel(
        _spmv_body,
        out_type=jax.ShapeDtypeStruct((_NC, _NP, cols), jnp.float32),
        mesh=plsc.VectorSubcoreMesh(core_axis_name="c", subcore_axis_name="s",
                                    num_cores=_NC, num_subcores=_NS),
        scratch_types=[
            pltpu.VMEM((_SEG, _CHUNK), jnp.int32),
            pltpu.VMEM((_SEG, _CHUNK), jnp.int32),
        ] + [pltpu.VMEM((_CHUNK, cols), jnp.float32)] * _NBUF + [
            pltpu.VMEM_SHARED((_NP, cols), jnp.float32),
        ] + [pltpu.SemaphoreType.DMA] * (2 * _NBUF),
        compiler_params=pltpu.CompilerParams(use_tc_tiling_on_sc=False),
    )


def _lif_block_body(tb, emit_tab, parts_ref, ext_ref, w_ref, vin_ref,
                    spk_ref, vseq_ref, vout_ref, tab_ref=None):
    irec = parts_ref[0] + parts_ref[1]
    irec_t = jnp.transpose(irec, (1, 0))
    w_row = w_ref[...]
    v = vin_ref[...]
    scaled = []
    for t in range(tb):
        i_tot = irec_t[4 * t : 4 * t + 4] + ext_ref[t]
        v = v * _DECAY + i_tot
        s = 1.0 / (1.0 + jnp.exp(-_BETA * (v - _V_TH)))
        spk_ref[t] = s
        v = v * (1.0 - s)
        vseq_ref[t] = v
        scaled.append(s * w_row)
    vout_ref[...] = v
    if emit_tab:
        zpad = jnp.zeros((_COLS - 4 * tb, v.shape[-1]), jnp.float32)
        tab_ref[...] = jnp.transpose(
            jnp.concatenate(scaled + [zpad], axis=0), (1, 0))


_NT = 5120


def _lif_block(tb, pcols, emit_tab):
    grid = (_NP // _NT,)
    out_specs = [
        pl.BlockSpec((tb, _B, _NT), lambda i: (0, 0, i)),
        pl.BlockSpec((tb, _B, _NT), lambda i: (0, 0, i)),
        pl.BlockSpec((_B, _NT), lambda i: (0, i)),
    ]
    out_shape = [
        jax.ShapeDtypeStruct((tb, _B, _NP), jnp.float32),
        jax.ShapeDtypeStruct((tb, _B, _NP), jnp.float32),
        jax.ShapeDtypeStruct((_B, _NP), jnp.float32),
    ]
    if emit_tab:
        out_specs.append(pl.BlockSpec((_NT, _COLS), lambda i: (i, 0)))
        out_shape.append(jax.ShapeDtypeStruct((_NP, _COLS), jnp.float32))
    return pl.pallas_call(
        functools.partial(_lif_block_body, tb, emit_tab),
        grid=grid,
        in_specs=[
            pl.BlockSpec((_NC, _NT, pcols), lambda i: (0, i, 0)),
            pl.BlockSpec((tb, _B, _NT), lambda i: (0, 0, i)),
            pl.BlockSpec((1, _NT), lambda i: (0, i)),
            pl.BlockSpec((_B, _NT), lambda i: (0, i)),
        ],
        out_specs=out_specs,
        out_shape=out_shape,
    )


_lif15 = _lif_block(15, _COLS, True)
_lif4 = _lif_block(4, 16, False)


def kernel(external_input, edge_index, edge_weight):
    del edge_weight
    ext = jnp.pad(external_input, ((0, 0), (0, 0), (0, _NP - _N)))
    pad_idx = (_N + jnp.arange(_EPAD - _E, dtype=jnp.int32) % (_NP - _N))
    srcp = jnp.concatenate([edge_index[0], pad_idx])
    srcp = srcp.reshape(_NW, _CHUNKS_PER_W, _CHUNK)
    dstp = jnp.concatenate([edge_index[1], pad_idx])
    dstp = dstp.reshape(_NW, _CHUNKS_PER_W, _CHUNK)
    w_row = jnp.where(jnp.arange(_NP) < _N_EXC, _W_EXC, _W_INH)
    w_row = w_row.astype(jnp.float32)[None, :]
    zero_tab = jnp.zeros((_NP, _COLS), jnp.float32)
    zero_tab16 = jnp.zeros((_NP, 16), jnp.float32)
    zero_parts = jnp.zeros((_NC, _NP, _COLS), jnp.float32)

    v = jnp.zeros((_B, _NP), jnp.float32)
    spks, vs = [], []

    s_blk, vseq, v, table = _lif15(zero_parts, ext[0:15], w_row, v)
    spks.append(s_blk)
    vs.append(vseq)

    spmv = _get_spmv(_COLS)
    for b in range(1, 4):
        parts = spmv(srcp, dstp, table, zero_tab)
        s_blk, vseq, v, table = _lif15(parts, ext[15 * b : 15 * b + 15],
                                       w_row, v)
        spks.append(s_blk)
        vs.append(vseq)

    parts = _get_spmv(16)(srcp, dstp, table[:, :16], zero_tab16)
    s_blk, vseq, v = _lif4(parts, ext[60:64], w_row, v)
    spks.append(s_blk)
    vs.append(vseq)

    spikes = jnp.concatenate(spks)[:, :, :_N]
    vout = jnp.concatenate(vs)[:, :, :_N]
    return spikes, vout
